# baseline (speedup 1.0000x reference)
import numpy as np

import jax
import jax.numpy as jnp
from jax.experimental import pallas as pl
from jax.experimental.pallas import tpu as pltpu

_B, _H, _W = 16, 512, 512
_N = float(_B * _H * _W)


def _bce_logits(x, t):
    return jnp.maximum(x, 0.0) - x * t + jnp.log1p(jnp.exp(-jnp.abs(x)))


def _reduce_body(preds_ref, gt_ref, gth_ref, mth_ref, out_ref):
    i = pl.program_id(0)

    def tile(it, acc):
        r = it * 8
        new = list(acc)
        for sub in range(4):
            c = sub * 128
            x0 = preds_ref[0, 0, pl.ds(r, 8), pl.ds(c, 128)]
            x1 = preds_ref[0, 1, pl.ds(r, 8), pl.ds(c, 128)]
            x2 = preds_ref[0, 2, pl.ds(r, 8), pl.ds(c, 128)]
            g = gt_ref[0, pl.ds(r, 8), pl.ds(c, 128)]
            gth = gth_ref[0, pl.ds(r, 8), pl.ds(c, 128)]
            mth = mth_ref[0, pl.ds(r, 8), pl.ds(c, 128)]

            e0 = jnp.exp(-jnp.abs(x0))
            t0 = 1.0 + e0
            sp0 = jnp.maximum(x0, 0.0) + jnp.log(t0)
            sig0 = jnp.where(x0 >= 0.0, 1.0, e0) / t0

            e2 = jnp.exp(-jnp.abs(x2))
            t2 = 1.0 + e2
            sp2 = jnp.maximum(x2, 0.0) + jnp.log(t2)
            sig2 = jnp.where(x2 >= 0.0, 1.0, e2) / t2

            d1 = jnp.abs((x1 - gth) * mth)

            vals = (sp0, sp0 * g, x0 * g, sig0, sig0 * g,
                    sp2 * g, x2 * g, sig2, sig2 * g, g, d1)
            new = [a + v for a, v in zip(new, vals)]
        return tuple(new)

    zeros = tuple(jnp.zeros((8, 128), jnp.float32) for _ in range(11))
    acc = jax.lax.fori_loop(0, _H // 8, tile, zeros)
    vals = tuple(jnp.sum(a) for a in acc)

    @pl.when(i == 0)
    def _init():
        for j, v in enumerate(vals):
            out_ref[0, j] = v

    @pl.when(i > 0)
    def _acc():
        for j, v in enumerate(vals):
            out_ref[0, j] += v


def _topk_body(x_ref, g_ref, k_ref, out_ref):
    k = k_ref[0, 0]
    nb = x_ref.shape[0]

    def masked_bce(j):
        x = x_ref[j]
        g = g_ref[j]
        return jnp.where(g == 0.0, _bce_logits(x, g), 0.0)

    def bit_step(it, t):
        t2 = t | jax.lax.shift_left(jnp.int32(1), 30 - it)

        def inner(j, c):
            bits = jax.lax.bitcast_convert_type(masked_bce(j), jnp.int32)
            return c + jnp.sum((bits >= t2).astype(jnp.float32))

        cnt = jax.lax.fori_loop(0, nb, inner, jnp.float32(0.0))
        return jnp.where(cnt >= k, t2, t)

    t = jax.lax.fori_loop(0, 31, bit_step, jnp.int32(0))
    kth = jax.lax.bitcast_convert_type(t, jnp.float32)

    def tail(j, carry):
        c, s = carry
        v = masked_bce(j)
        m = (v > kth).astype(jnp.float32)
        return (c + jnp.sum(m), s + jnp.sum(v * m))

    cnt_gt, sum_gt = jax.lax.fori_loop(
        0, nb, tail, (jnp.float32(0.0), jnp.float32(0.0)))
    out_ref[0, 0] = sum_gt + (k - cnt_gt) * kth


def _sharded_loss(preds, gt_prob, gt_thresh, mask_thresh):
    nb = preds.shape[0]
    sums = pl.pallas_call(
        _reduce_body,
        grid=(nb,),
        in_specs=[
            pl.BlockSpec((1, 3, _H, _W), lambda i: (i, 0, 0, 0)),
            pl.BlockSpec((1, _H, _W), lambda i: (i, 0, 0)),
            pl.BlockSpec((1, _H, _W), lambda i: (i, 0, 0)),
            pl.BlockSpec((1, _H, _W), lambda i: (i, 0, 0)),
        ],
        out_specs=pl.BlockSpec(
            (1, 16), lambda i: (0, 0), memory_space=pltpu.SMEM),
        out_shape=jax.ShapeDtypeStruct((1, 16), jnp.float32),
    )(preds, gt_prob, gt_thresh, mask_thresh)
    sums = jax.lax.psum(sums, "b")

    s = sums[0]
    sp0_sum, sp0g_sum, x0g_sum = s[0], s[1], s[2]
    sig0_sum, inter0 = s[3], s[4]
    sp2g_sum, x2g_sum, sig2_sum, inter2 = s[5], s[6], s[7], s[8]
    g_sum, abs_sum = s[9], s[10]

    pos_cnt = g_sum
    neg_cnt = _N - g_sum
    pos_bce = sp0g_sum - x0g_sum
    neg_bce = sp0_sum - sp0g_sum
    pos_bce2 = sp2g_sum - x2g_sum

    num_negative = jnp.floor(jnp.minimum(pos_cnt * 3.0, neg_cnt))

    def _common(_):
        return neg_bce

    def _rare(_):
        x_full = jax.lax.all_gather(preds[:, 0], "b", axis=0, tiled=True)
        g_full = jax.lax.all_gather(gt_prob, "b", axis=0, tiled=True)
        return pl.pallas_call(
            _topk_body,
            in_specs=[
                pl.BlockSpec(memory_space=pltpu.VMEM),
                pl.BlockSpec(memory_space=pltpu.VMEM),
                pl.BlockSpec(memory_space=pltpu.SMEM),
            ],
            out_specs=pl.BlockSpec(memory_space=pltpu.SMEM),
            out_shape=jax.ShapeDtypeStruct((1, 1), jnp.float32),
        )(x_full, g_full, num_negative.reshape(1, 1))[0, 0]

    topk_sum = jax.lax.cond(pos_cnt * 3.0 >= neg_cnt, _common, _rare, None)

    positive_loss = pos_bce / (pos_cnt + 1e-06)
    negative_loss_mean = topk_sum / num_negative
    dice0 = 1.0 - (2.0 * inter0 + 1.0) / (sig0_sum + g_sum + 1.0)
    loss_prob = positive_loss + negative_loss_mean + dice0

    loss_thresh = abs_sum / _N

    dice2 = 1.0 - (2.0 * inter2 + 1.0) / (sig2_sum + g_sum + 1.0)
    loss_binary = pos_bce2 / _N + dice2

    return loss_prob + 10.0 * loss_thresh + loss_binary


def kernel(preds, gt_prob, gt_thresh, mask_prob, mask_thresh):
    del mask_prob
    devs = jax.devices()
    ndev = 2 if len(devs) >= 2 and _B % 2 == 0 else 1
    mesh = jax.sharding.Mesh(np.array(devs[:ndev]), ("b",))
    spec = jax.sharding.PartitionSpec("b")
    f = jax.shard_map(
        _sharded_loss,
        mesh=mesh,
        in_specs=(spec, spec, spec, spec),
        out_specs=jax.sharding.PartitionSpec(),
        check_vma=False,
    )
    return f(preds, gt_prob, gt_thresh, mask_thresh)


# speedup vs baseline: 8.7133x; 8.7133x over previous
import numpy as np

import jax
import jax.numpy as jnp
from jax.experimental import pallas as pl
from jax.experimental.pallas import tpu as pltpu

_B, _H, _W = 16, 512, 512
_N = float(_B * _H * _W)


def _bce_logits(x, t):
    return jnp.maximum(x, 0.0) - x * t + jnp.log1p(jnp.exp(-jnp.abs(x)))


def _reduce_body(preds_ref, gt_ref, gth_ref, mth_ref, out_ref, acc_ref):
    i = pl.program_id(0)

    def tile(it, acc):
        b = it // (_H // 8)
        r = (it % (_H // 8)) * 8
        new = list(acc)
        for sub in range(4):
            c = sub * 128
            x0 = preds_ref[b, 0, pl.ds(r, 8), pl.ds(c, 128)]
            x1 = preds_ref[b, 1, pl.ds(r, 8), pl.ds(c, 128)]
            x2 = preds_ref[b, 2, pl.ds(r, 8), pl.ds(c, 128)]
            g = gt_ref[b, pl.ds(r, 8), pl.ds(c, 128)]
            gth = gth_ref[b, pl.ds(r, 8), pl.ds(c, 128)]
            mth = mth_ref[b, pl.ds(r, 8), pl.ds(c, 128)]

            sp0 = jnp.maximum(x0, 0.0) + jnp.log(1.0 + jnp.exp(-jnp.abs(x0)))
            h0 = jnp.tanh(x0 * 0.5)
            sp2 = jnp.maximum(x2, 0.0) + jnp.log(1.0 + jnp.exp(-jnp.abs(x2)))
            h2 = jnp.tanh(x2 * 0.5)
            d1 = jnp.abs((x1 - gth) * mth)

            vals = (sp0, sp0 * g, x0 * g, h0, h0 * g,
                    sp2 * g, x2 * g, h2, h2 * g, g, d1)
            new = [a + v for a, v in zip(new, vals)]
        return tuple(new)

    zeros = tuple(jnp.zeros((8, 128), jnp.float32) for _ in range(11))
    acc = jax.lax.fori_loop(0, 2 * (_H // 8), tile, zeros)

    @pl.when(i == 0)
    def _init():
        for j, a in enumerate(acc):
            acc_ref[j] = a

    @pl.when(i > 0)
    def _acc():
        for j, a in enumerate(acc):
            acc_ref[j] += a

    @pl.when(i == pl.num_programs(0) - 1)
    def _final():
        for j in range(11):
            out_ref[0, j] = jnp.sum(acc_ref[j])


def _topk_body(x_ref, g_ref, k_ref, out_ref):
    k = k_ref[0, 0]
    nb = x_ref.shape[0]

    def masked_bce(j):
        x = x_ref[j]
        g = g_ref[j]
        return jnp.where(g == 0.0, _bce_logits(x, g), 0.0)

    def bit_step(it, t):
        t2 = t | jax.lax.shift_left(jnp.int32(1), 30 - it)

        def inner(j, c):
            bits = jax.lax.bitcast_convert_type(masked_bce(j), jnp.int32)
            return c + jnp.sum((bits >= t2).astype(jnp.float32))

        cnt = jax.lax.fori_loop(0, nb, inner, jnp.float32(0.0))
        return jnp.where(cnt >= k, t2, t)

    t = jax.lax.fori_loop(0, 31, bit_step, jnp.int32(0))
    kth = jax.lax.bitcast_convert_type(t, jnp.float32)

    def tail(j, carry):
        c, s = carry
        v = masked_bce(j)
        m = (v > kth).astype(jnp.float32)
        return (c + jnp.sum(m), s + jnp.sum(v * m))

    cnt_gt, sum_gt = jax.lax.fori_loop(
        0, nb, tail, (jnp.float32(0.0), jnp.float32(0.0)))
    out_ref[0, 0] = sum_gt + (k - cnt_gt) * kth


def _sharded_loss(preds, gt_prob, gt_thresh, mask_thresh):
    nb = preds.shape[0]
    sums = pl.pallas_call(
        _reduce_body,
        grid=(nb // 2,),
        in_specs=[
            pl.BlockSpec((2, 3, _H, _W), lambda i: (i, 0, 0, 0)),
            pl.BlockSpec((2, _H, _W), lambda i: (i, 0, 0)),
            pl.BlockSpec((2, _H, _W), lambda i: (i, 0, 0)),
            pl.BlockSpec((2, _H, _W), lambda i: (i, 0, 0)),
        ],
        out_specs=pl.BlockSpec(
            (1, 16), lambda i: (0, 0), memory_space=pltpu.SMEM),
        out_shape=jax.ShapeDtypeStruct((1, 16), jnp.float32),
        scratch_shapes=[pltpu.VMEM((11, 8, 128), jnp.float32)],
    )(preds, gt_prob, gt_thresh, mask_thresh)
    sums = jax.lax.psum(sums, "b")

    s = sums[0]
    sp0_sum, sp0g_sum, x0g_sum = s[0], s[1], s[2]
    h0_sum, h0g_sum = s[3], s[4]
    sp2g_sum, x2g_sum, h2_sum, h2g_sum = s[5], s[6], s[7], s[8]
    g_sum, abs_sum = s[9], s[10]

    sig0_sum = 0.5 * h0_sum + 0.5 * _N
    inter0 = 0.5 * h0g_sum + 0.5 * g_sum
    sig2_sum = 0.5 * h2_sum + 0.5 * _N
    inter2 = 0.5 * h2g_sum + 0.5 * g_sum

    pos_cnt = g_sum
    neg_cnt = _N - g_sum
    pos_bce = sp0g_sum - x0g_sum
    neg_bce = sp0_sum - sp0g_sum
    pos_bce2 = sp2g_sum - x2g_sum

    num_negative = jnp.floor(jnp.minimum(pos_cnt * 3.0, neg_cnt))

    def _common(_):
        return neg_bce

    def _rare(_):
        x_full = jax.lax.all_gather(preds[:, 0], "b", axis=0, tiled=True)
        g_full = jax.lax.all_gather(gt_prob, "b", axis=0, tiled=True)
        return pl.pallas_call(
            _topk_body,
            in_specs=[
                pl.BlockSpec(memory_space=pltpu.VMEM),
                pl.BlockSpec(memory_space=pltpu.VMEM),
                pl.BlockSpec(memory_space=pltpu.SMEM),
            ],
            out_specs=pl.BlockSpec(memory_space=pltpu.SMEM),
            out_shape=jax.ShapeDtypeStruct((1, 1), jnp.float32),
        )(x_full, g_full, num_negative.reshape(1, 1))[0, 0]

    topk_sum = jax.lax.cond(pos_cnt * 3.0 >= neg_cnt, _common, _rare, None)

    positive_loss = pos_bce / (pos_cnt + 1e-06)
    negative_loss_mean = topk_sum / num_negative
    dice0 = 1.0 - (2.0 * inter0 + 1.0) / (sig0_sum + g_sum + 1.0)
    loss_prob = positive_loss + negative_loss_mean + dice0

    loss_thresh = abs_sum / _N

    dice2 = 1.0 - (2.0 * inter2 + 1.0) / (sig2_sum + g_sum + 1.0)
    loss_binary = pos_bce2 / _N + dice2

    return loss_prob + 10.0 * loss_thresh + loss_binary


def kernel(preds, gt_prob, gt_thresh, mask_prob, mask_thresh):
    del mask_prob
    mesh = jax.sharding.Mesh(np.array(jax.devices()[:1]), ("b",))
    spec = jax.sharding.PartitionSpec()
    f = jax.shard_map(
        _sharded_loss,
        mesh=mesh,
        in_specs=(spec, spec, spec, spec),
        out_specs=jax.sharding.PartitionSpec(),
        check_vma=False,
    )
    return f(preds, gt_prob, gt_thresh, mask_thresh)


# speedup vs baseline: 9.7947x; 1.1241x over previous
import numpy as np

import jax
import jax.numpy as jnp
from jax.experimental import pallas as pl
from jax.experimental.pallas import tpu as pltpu

_B, _H, _W = 16, 512, 512
_N = float(_B * _H * _W)


def _bce_logits(x, t):
    return jnp.maximum(x, 0.0) - x * t + jnp.log1p(jnp.exp(-jnp.abs(x)))


_RROWS = 256
_NREG = _B * (_H // _RROWS)
_NBUF = 4


def _region_copies(preds_ref, gt_ref, gth_ref, mth_ref,
                   bx_ref, bg_ref, bgth_ref, bmth_ref, sem_ref, i, slot):
    b = i // (_H // _RROWS)
    r = (i % (_H // _RROWS)) * _RROWS
    return (
        pltpu.make_async_copy(
            preds_ref.at[b, :, pl.ds(r, _RROWS), :], bx_ref.at[slot],
            sem_ref.at[slot, 0]),
        pltpu.make_async_copy(
            gt_ref.at[b, pl.ds(r, _RROWS), :], bg_ref.at[slot],
            sem_ref.at[slot, 1]),
        pltpu.make_async_copy(
            gth_ref.at[b, pl.ds(r, _RROWS), :], bgth_ref.at[slot],
            sem_ref.at[slot, 2]),
        pltpu.make_async_copy(
            mth_ref.at[b, pl.ds(r, _RROWS), :], bmth_ref.at[slot],
            sem_ref.at[slot, 3]),
    )


def _reduce_body(preds_ref, gt_ref, gth_ref, mth_ref, out_ref,
                 bx_ref, bg_ref, bgth_ref, bmth_ref, sem_ref):
    def start(i, slot):
        for cp in _region_copies(preds_ref, gt_ref, gth_ref, mth_ref,
                                 bx_ref, bg_ref, bgth_ref, bmth_ref,
                                 sem_ref, i, slot):
            cp.start()

    def wait(i, slot):
        for cp in _region_copies(preds_ref, gt_ref, gth_ref, mth_ref,
                                 bx_ref, bg_ref, bgth_ref, bmth_ref,
                                 sem_ref, i, slot):
            cp.wait()

    for s in range(_NBUF):
        start(s, s)

    def tile(it, carry):
        slot, acc = carry
        r = it * 8
        new = list(acc)
        for sub in range(4):
            c = sub * 128
            x0 = bx_ref[slot, 0, pl.ds(r, 8), pl.ds(c, 128)]
            x1 = bx_ref[slot, 1, pl.ds(r, 8), pl.ds(c, 128)]
            x2 = bx_ref[slot, 2, pl.ds(r, 8), pl.ds(c, 128)]
            g = bg_ref[slot, pl.ds(r, 8), pl.ds(c, 128)]
            gth = bgth_ref[slot, pl.ds(r, 8), pl.ds(c, 128)]
            mth = bmth_ref[slot, pl.ds(r, 8), pl.ds(c, 128)]

            sp0 = jnp.maximum(x0, 0.0) + jnp.log(1.0 + jnp.exp(-jnp.abs(x0)))
            h0 = jnp.tanh(x0 * 0.5)
            sp2 = jnp.maximum(x2, 0.0) + jnp.log(1.0 + jnp.exp(-jnp.abs(x2)))
            h2 = jnp.tanh(x2 * 0.5)
            d1 = jnp.abs((x1 - gth) * mth)

            vals = (sp0, sp0 * g, x0 * g, h0, h0 * g,
                    sp2 * g, x2 * g, h2, h2 * g, g, d1)
            new = [a + v for a, v in zip(new, vals)]
        return (slot, tuple(new))

    def region(i, acc):
        slot = jax.lax.rem(i, _NBUF)
        wait(i, slot)
        _, acc = jax.lax.fori_loop(0, _RROWS // 8, tile, (slot, acc))

        @pl.when(i + _NBUF < _NREG)
        def _():
            start(i + _NBUF, slot)

        return acc

    zeros = tuple(jnp.zeros((8, 128), jnp.float32) for _ in range(11))
    acc = jax.lax.fori_loop(0, _NREG, region, zeros)

    for j, a in enumerate(acc):
        out_ref[0, j] = jnp.sum(a)


def _topk_body(x_ref, g_ref, k_ref, out_ref):
    k = k_ref[0, 0]
    nb = x_ref.shape[0]

    def masked_bce(j):
        x = x_ref[j]
        g = g_ref[j]
        return jnp.where(g == 0.0, _bce_logits(x, g), 0.0)

    def bit_step(it, t):
        t2 = t | jax.lax.shift_left(jnp.int32(1), 30 - it)

        def inner(j, c):
            bits = jax.lax.bitcast_convert_type(masked_bce(j), jnp.int32)
            return c + jnp.sum((bits >= t2).astype(jnp.float32))

        cnt = jax.lax.fori_loop(0, nb, inner, jnp.float32(0.0))
        return jnp.where(cnt >= k, t2, t)

    t = jax.lax.fori_loop(0, 31, bit_step, jnp.int32(0))
    kth = jax.lax.bitcast_convert_type(t, jnp.float32)

    def tail(j, carry):
        c, s = carry
        v = masked_bce(j)
        m = (v > kth).astype(jnp.float32)
        return (c + jnp.sum(m), s + jnp.sum(v * m))

    cnt_gt, sum_gt = jax.lax.fori_loop(
        0, nb, tail, (jnp.float32(0.0), jnp.float32(0.0)))
    out_ref[0, 0] = sum_gt + (k - cnt_gt) * kth


def _sharded_loss(preds, gt_prob, gt_thresh, mask_thresh):
    sums = pl.pallas_call(
        _reduce_body,
        in_specs=[
            pl.BlockSpec(memory_space=pltpu.HBM),
            pl.BlockSpec(memory_space=pltpu.HBM),
            pl.BlockSpec(memory_space=pltpu.HBM),
            pl.BlockSpec(memory_space=pltpu.HBM),
        ],
        out_specs=pl.BlockSpec(memory_space=pltpu.SMEM),
        out_shape=jax.ShapeDtypeStruct((1, 16), jnp.float32),
        scratch_shapes=[
            pltpu.VMEM((_NBUF, 3, _RROWS, _W), jnp.float32),
            pltpu.VMEM((_NBUF, _RROWS, _W), jnp.float32),
            pltpu.VMEM((_NBUF, _RROWS, _W), jnp.float32),
            pltpu.VMEM((_NBUF, _RROWS, _W), jnp.float32),
            pltpu.SemaphoreType.DMA((_NBUF, 4)),
        ],
    )(preds, gt_prob, gt_thresh, mask_thresh)
    sums = jax.lax.psum(sums, "b")

    s = sums[0]
    sp0_sum, sp0g_sum, x0g_sum = s[0], s[1], s[2]
    h0_sum, h0g_sum = s[3], s[4]
    sp2g_sum, x2g_sum, h2_sum, h2g_sum = s[5], s[6], s[7], s[8]
    g_sum, abs_sum = s[9], s[10]

    sig0_sum = 0.5 * h0_sum + 0.5 * _N
    inter0 = 0.5 * h0g_sum + 0.5 * g_sum
    sig2_sum = 0.5 * h2_sum + 0.5 * _N
    inter2 = 0.5 * h2g_sum + 0.5 * g_sum

    pos_cnt = g_sum
    neg_cnt = _N - g_sum
    pos_bce = sp0g_sum - x0g_sum
    neg_bce = sp0_sum - sp0g_sum
    pos_bce2 = sp2g_sum - x2g_sum

    num_negative = jnp.floor(jnp.minimum(pos_cnt * 3.0, neg_cnt))

    def _common(_):
        return neg_bce

    def _rare(_):
        x_full = jax.lax.all_gather(preds[:, 0], "b", axis=0, tiled=True)
        g_full = jax.lax.all_gather(gt_prob, "b", axis=0, tiled=True)
        return pl.pallas_call(
            _topk_body,
            in_specs=[
                pl.BlockSpec(memory_space=pltpu.VMEM),
                pl.BlockSpec(memory_space=pltpu.VMEM),
                pl.BlockSpec(memory_space=pltpu.SMEM),
            ],
            out_specs=pl.BlockSpec(memory_space=pltpu.SMEM),
            out_shape=jax.ShapeDtypeStruct((1, 1), jnp.float32),
        )(x_full, g_full, num_negative.reshape(1, 1))[0, 0]

    topk_sum = jax.lax.cond(pos_cnt * 3.0 >= neg_cnt, _common, _rare, None)

    positive_loss = pos_bce / (pos_cnt + 1e-06)
    negative_loss_mean = topk_sum / num_negative
    dice0 = 1.0 - (2.0 * inter0 + 1.0) / (sig0_sum + g_sum + 1.0)
    loss_prob = positive_loss + negative_loss_mean + dice0

    loss_thresh = abs_sum / _N

    dice2 = 1.0 - (2.0 * inter2 + 1.0) / (sig2_sum + g_sum + 1.0)
    loss_binary = pos_bce2 / _N + dice2

    return loss_prob + 10.0 * loss_thresh + loss_binary


def kernel(preds, gt_prob, gt_thresh, mask_prob, mask_thresh):
    del mask_prob
    mesh = jax.sharding.Mesh(np.array(jax.devices()[:1]), ("b",))
    spec = jax.sharding.PartitionSpec()
    f = jax.shard_map(
        _sharded_loss,
        mesh=mesh,
        in_specs=(spec, spec, spec, spec),
        out_specs=jax.sharding.PartitionSpec(),
        check_vma=False,
    )
    return f(preds, gt_prob, gt_thresh, mask_thresh)
